# baseline (speedup 1.0000x reference)
import functools

import jax
import jax.numpy as jnp
from jax import lax
from jax.experimental import pallas as pl
from jax.experimental.pallas import tpu as pltpu
from jax.experimental.pallas import tpu_sc as plsc

_NUM_CORES = 2
_NUM_SUBCORES = 16
_NW = _NUM_CORES * _NUM_SUBCORES
_SUBLANES = 8
_G = 16


def _sc_gather(table3, q2, s2, batch, dim):
    b_per_w = batch // _NW
    n_groups = b_per_w // _G

    mesh = plsc.VectorSubcoreMesh(core_axis_name="c", subcore_axis_name="s")

    @functools.partial(
        pl.kernel,
        out_type=jax.ShapeDtypeStruct((batch, dim), jnp.float32),
        mesh=mesh,
        scratch_types=[
            pltpu.VMEM((b_per_w,), jnp.int32),
            pltpu.VMEM((b_per_w,), jnp.int32),
            pltpu.VMEM((b_per_w, dim), jnp.float32),
            pltpu.SemaphoreType.DMA,
        ],
    )
    def gather_kernel(table_hbm, q_hbm, s_hbm, out_hbm, q_v, s_v, rows_v, sem):
        wid = lax.axis_index("s") * _NUM_CORES + lax.axis_index("c")
        base = wid * b_per_w
        pltpu.sync_copy(q_hbm.at[wid], q_v)
        pltpu.sync_copy(s_hbm.at[wid], s_v)

        def body(g, _):
            qv = q_v[pl.ds(g * _G, _G)]
            sv = s_v[pl.ds(g * _G, _G)]
            for l in range(_G):
                pltpu.async_copy(
                    table_hbm.at[qv[l], sv[l]],
                    rows_v.at[g * _G + l],
                    sem,
                )
            return _

        lax.fori_loop(0, n_groups, body, None)
        pltpu.make_async_copy(out_hbm.at[pl.ds(base, b_per_w)], rows_v, sem).wait()
        pltpu.sync_copy(rows_v, out_hbm.at[pl.ds(base, b_per_w)])

    return gather_kernel(table3, q2, s2)


def _tc_film(x, enc, W, b2d, blk):
    batch, dim = x.shape

    def film_kernel(x_ref, enc_ref, w_ref, b_ref, out_ref):
        gb = (
            jnp.dot(
                enc_ref[...],
                w_ref[...],
                preferred_element_type=jnp.float32,
                precision=lax.Precision.HIGHEST,
            )
            + b_ref[...]
        )
        gamma = gb[:, :dim]
        beta = gb[:, dim:]
        out_ref[...] = x_ref[...] * (1.0 + gamma) + beta

    return pl.pallas_call(
        film_kernel,
        grid=(batch // blk,),
        in_specs=[
            pl.BlockSpec((blk, dim), lambda i: (i, 0)),
            pl.BlockSpec((blk, dim), lambda i: (i, 0)),
            pl.BlockSpec(W.shape, lambda i: (0, 0)),
            pl.BlockSpec(b2d.shape, lambda i: (0, 0)),
        ],
        out_specs=pl.BlockSpec((blk, dim), lambda i: (i, 0)),
        out_shape=jax.ShapeDtypeStruct((batch, dim), jnp.float32),
    )(x, enc, W, b2d)


@jax.jit
def kernel(x, label, emb_table, W, b):
    batch, dim = x.shape
    rows = emb_table.shape[0]
    idx = label.astype(jnp.int32)
    table3 = emb_table.reshape(rows // _SUBLANES, _SUBLANES, dim)
    b2d = b.reshape(1, -1)
    h = batch // 2
    outs = []
    for p in range(2):
        idx_h = lax.slice(idx, (p * h,), ((p + 1) * h,))
        q2 = (idx_h // _SUBLANES).reshape(_NW, h // _NW)
        s2 = (idx_h % _SUBLANES).reshape(_NW, h // _NW)
        enc = _sc_gather(table3, q2, s2, h, dim)
        x_h = lax.slice(x, (p * h, 0), ((p + 1) * h, dim))
        outs.append(_tc_film(x_h, enc, W, b2d, blk=2048))
    return jnp.concatenate(outs, axis=0)


# speedup vs baseline: 1.0200x; 1.0200x over previous
import functools

import jax
import jax.numpy as jnp
from jax import lax
from jax.experimental import pallas as pl
from jax.experimental.pallas import tpu as pltpu
from jax.experimental.pallas import tpu_sc as plsc

_NUM_CORES = 2
_NUM_SUBCORES = 16
_NW = _NUM_CORES * _NUM_SUBCORES
_SUBLANES = 8
_G = 16


def _sc_gather(table3, q2, s2, batch, dim):
    b_per_w = batch // _NW
    n_groups = b_per_w // _G

    mesh = plsc.VectorSubcoreMesh(core_axis_name="c", subcore_axis_name="s")

    @functools.partial(
        pl.kernel,
        out_type=jax.ShapeDtypeStruct((batch, dim), jnp.float32),
        mesh=mesh,
        scratch_types=[
            pltpu.VMEM((b_per_w,), jnp.int32),
            pltpu.VMEM((b_per_w,), jnp.int32),
            pltpu.VMEM((b_per_w, dim), jnp.float32),
            pltpu.SemaphoreType.DMA,
        ],
    )
    def gather_kernel(table_hbm, q_hbm, s_hbm, out_hbm, q_v, s_v, rows_v, sem):
        wid = lax.axis_index("s") * _NUM_CORES + lax.axis_index("c")
        base = wid * b_per_w
        pltpu.sync_copy(q_hbm.at[wid], q_v)
        pltpu.sync_copy(s_hbm.at[wid], s_v)

        def body(g, _):
            qv = q_v[pl.ds(g * _G, _G)]
            sv = s_v[pl.ds(g * _G, _G)]
            for l in range(_G):
                pltpu.async_copy(
                    table_hbm.at[qv[l], sv[l]],
                    rows_v.at[g * _G + l],
                    sem,
                )
            return _

        lax.fori_loop(0, n_groups, body, None)
        pltpu.make_async_copy(out_hbm.at[pl.ds(base, b_per_w)], rows_v, sem).wait()
        pltpu.sync_copy(rows_v, out_hbm.at[pl.ds(base, b_per_w)])

    return gather_kernel(table3, q2, s2)


def _tc_film(x, enc, W, b2d, blk):
    batch, dim = x.shape

    def film_kernel(x_ref, enc_ref, w_ref, b_ref, out_ref):
        gb = (
            jnp.dot(
                enc_ref[...],
                w_ref[...],
                preferred_element_type=jnp.float32,
                precision=lax.Precision.HIGHEST,
            )
            + b_ref[...]
        )
        gamma = gb[:, :dim]
        beta = gb[:, dim:]
        out_ref[...] = x_ref[...] * (1.0 + gamma) + beta

    return pl.pallas_call(
        film_kernel,
        grid=(batch // blk,),
        in_specs=[
            pl.BlockSpec((blk, dim), lambda i: (i, 0)),
            pl.BlockSpec((blk, dim), lambda i: (i, 0)),
            pl.BlockSpec(W.shape, lambda i: (0, 0)),
            pl.BlockSpec(b2d.shape, lambda i: (0, 0)),
        ],
        out_specs=pl.BlockSpec((blk, dim), lambda i: (i, 0)),
        out_shape=jax.ShapeDtypeStruct((batch, dim), jnp.float32),
    )(x, enc, W, b2d)


@jax.jit
def kernel(x, label, emb_table, W, b):
    batch, dim = x.shape
    rows = emb_table.shape[0]
    idx = label.astype(jnp.int32)
    q2 = (idx // _SUBLANES).reshape(_NW, batch // _NW)
    s2 = (idx % _SUBLANES).reshape(_NW, batch // _NW)
    table3 = emb_table.reshape(rows // _SUBLANES, _SUBLANES, dim)
    enc = _sc_gather(table3, q2, s2, batch, dim)
    return _tc_film(x, enc, W, b.reshape(1, -1), blk=2048)


# speedup vs baseline: 1.0695x; 1.0485x over previous
import functools

import jax
import jax.numpy as jnp
from jax import lax
from jax.experimental import pallas as pl
from jax.experimental.pallas import tpu as pltpu
from jax.experimental.pallas import tpu_sc as plsc

_NUM_CORES = 2
_NUM_SUBCORES = 16
_NW = _NUM_CORES * _NUM_SUBCORES
_SUBLANES = 8
_G = 16


def _sc_gather(table3, q2, s2, batch, dim):
    b_per_w = batch // _NW
    n_groups = b_per_w // _G

    mesh = plsc.VectorSubcoreMesh(core_axis_name="c", subcore_axis_name="s")

    @functools.partial(
        pl.kernel,
        out_type=jax.ShapeDtypeStruct((batch, dim), jnp.float32),
        mesh=mesh,
        scratch_types=[
            pltpu.VMEM((b_per_w,), jnp.int32),
            pltpu.VMEM((b_per_w,), jnp.int32),
            pltpu.VMEM((b_per_w, dim), jnp.float32),
            pltpu.SemaphoreType.DMA,
        ],
    )
    def gather_kernel(table_hbm, q_hbm, s_hbm, out_hbm, q_v, s_v, rows_v, sem):
        wid = lax.axis_index("s") * _NUM_CORES + lax.axis_index("c")
        base = wid * b_per_w
        pltpu.sync_copy(q_hbm.at[wid], q_v)
        pltpu.sync_copy(s_hbm.at[wid], s_v)

        def body(g, _):
            qv = q_v[pl.ds(g * _G, _G)]
            sv = s_v[pl.ds(g * _G, _G)]
            for l in range(_G):
                pltpu.async_copy(
                    table_hbm.at[qv[l], sv[l]],
                    rows_v.at[g * _G + l],
                    sem,
                )
            return _

        lax.fori_loop(0, n_groups, body, None)
        pltpu.make_async_copy(out_hbm.at[pl.ds(base, b_per_w)], rows_v, sem).wait()
        pltpu.sync_copy(rows_v, out_hbm.at[pl.ds(base, b_per_w)])

    return gather_kernel(table3, q2, s2)


def _tc_film(xT, enc, W, b2d, blk):
    dim, batch = xT.shape

    def film_kernel(enc_ref, xT_ref, w_ref, b_ref, outT_ref):
        gb = (
            jnp.dot(
                enc_ref[...],
                w_ref[...],
                preferred_element_type=jnp.float32,
                precision=lax.Precision.HIGHEST,
            )
            + b_ref[...]
        )
        gbT = gb.T
        outT_ref[...] = xT_ref[...] * (1.0 + gbT[:dim, :]) + gbT[dim:, :]

    return pl.pallas_call(
        film_kernel,
        grid=(batch // blk,),
        in_specs=[
            pl.BlockSpec((blk, dim), lambda i: (i, 0)),
            pl.BlockSpec((dim, blk), lambda i: (0, i)),
            pl.BlockSpec(W.shape, lambda i: (0, 0)),
            pl.BlockSpec(b2d.shape, lambda i: (0, 0)),
        ],
        out_specs=pl.BlockSpec((dim, blk), lambda i: (0, i)),
        out_shape=jax.ShapeDtypeStruct((dim, batch), jnp.float32),
    )(enc, xT, W, b2d)


@jax.jit
def kernel(x, label, emb_table, W, b):
    batch, dim = x.shape
    rows = emb_table.shape[0]
    idx = label.astype(jnp.int32)
    q2 = (idx // _SUBLANES).reshape(_NW, batch // _NW)
    s2 = (idx % _SUBLANES).reshape(_NW, batch // _NW)
    table3 = emb_table.reshape(rows // _SUBLANES, _SUBLANES, dim)
    enc = _sc_gather(table3, q2, s2, batch, dim)
    outT = _tc_film(x.T, enc, W, b.reshape(1, -1), blk=2048)
    return outT.T


# speedup vs baseline: 1.0747x; 1.0048x over previous
import functools

import jax
import jax.numpy as jnp
from jax import lax
from jax.experimental import pallas as pl
from jax.experimental.pallas import tpu as pltpu
from jax.experimental.pallas import tpu_sc as plsc

_NUM_CORES = 2
_NUM_SUBCORES = 16
_NW = _NUM_CORES * _NUM_SUBCORES
_SUBLANES = 8
_G = 16


def _sc_gather(table3, q2, s2, batch, dim):
    b_per_w = batch // _NW
    n_groups = b_per_w // _G

    mesh = plsc.VectorSubcoreMesh(core_axis_name="c", subcore_axis_name="s")

    @functools.partial(
        pl.kernel,
        out_type=jax.ShapeDtypeStruct((batch, dim), jnp.float32),
        mesh=mesh,
        scratch_types=[
            pltpu.VMEM((b_per_w,), jnp.int32),
            pltpu.VMEM((b_per_w,), jnp.int32),
            pltpu.VMEM((b_per_w, dim), jnp.float32),
            pltpu.SemaphoreType.DMA,
        ],
    )
    def gather_kernel(table_hbm, q_hbm, s_hbm, out_hbm, q_v, s_v, rows_v, sem):
        wid = lax.axis_index("s") * _NUM_CORES + lax.axis_index("c")
        base = wid * b_per_w
        pltpu.sync_copy(q_hbm.at[wid], q_v)
        pltpu.sync_copy(s_hbm.at[wid], s_v)

        def body(g, _):
            qv = q_v[pl.ds(g * _G, _G)]
            sv = s_v[pl.ds(g * _G, _G)]
            for l in range(_G):
                pltpu.async_copy(
                    table_hbm.at[qv[l], sv[l]],
                    rows_v.at[g * _G + l],
                    sem,
                )
            return _

        lax.fori_loop(0, n_groups, body, None)
        pltpu.make_async_copy(out_hbm.at[pl.ds(base, b_per_w)], rows_v, sem).wait()
        pltpu.sync_copy(rows_v, out_hbm.at[pl.ds(base, b_per_w)])

    return gather_kernel(table3, q2, s2)


def _tc_film(xT, enc, W, b2d, blk):
    dim, batch = xT.shape

    def film_kernel(enc_ref, xT_ref, w_ref, b_ref, outT_ref):
        gb = (
            jnp.dot(
                enc_ref[...],
                w_ref[...],
                preferred_element_type=jnp.float32,
                precision=lax.Precision.HIGHEST,
            )
            + b_ref[...]
        )
        gbT = gb.T
        outT_ref[...] = xT_ref[...] * (1.0 + gbT[:dim, :]) + gbT[dim:, :]

    return pl.pallas_call(
        film_kernel,
        grid=(batch // blk,),
        in_specs=[
            pl.BlockSpec((blk, dim), lambda i: (i, 0)),
            pl.BlockSpec((dim, blk), lambda i: (0, i)),
            pl.BlockSpec(W.shape, lambda i: (0, 0)),
            pl.BlockSpec(b2d.shape, lambda i: (0, 0)),
        ],
        out_specs=pl.BlockSpec((dim, blk), lambda i: (0, i)),
        out_shape=jax.ShapeDtypeStruct((dim, batch), jnp.float32),
    )(enc, xT, W, b2d)


@jax.jit
def kernel(x, label, emb_table, W, b):
    batch, dim = x.shape
    rows = emb_table.shape[0]
    idx = label.astype(jnp.int32)
    q2 = (idx // _SUBLANES).reshape(_NW, batch // _NW)
    s2 = (idx % _SUBLANES).reshape(_NW, batch // _NW)
    table3 = emb_table.reshape(rows // _SUBLANES, _SUBLANES, dim)
    enc = _sc_gather(table3, q2, s2, batch, dim)
    outT = _tc_film(x.T, enc, W, b.reshape(1, -1), blk=4096)
    return outT.T
